# Initial kernel scaffold; baseline (speedup 1.0000x reference)
import jax
import jax.numpy as jnp
from jax.experimental import pallas as pl


def kernel(x, edge_index, edge_attr, batch_id, emb_table, params, lin_W, lin_b):
    raise NotImplementedError("write your pallas kernel here")


# speedup vs baseline: 6.3829x; 6.3829x over previous
import functools

import jax
import jax.numpy as jnp
from jax import lax
from jax.experimental import pallas as pl
from jax.experimental.pallas import tpu as pltpu
from jax.experimental.pallas import tpu_sc as plsc

N = 10000
E = 320000
G = 16
EC = 16
STATIONS = 535
SPAD = 640

NC = 2
NS = 16
NW = NC * NS
EPT = E // NW
ECH = 80
NCHUNK = EPT // ECH
NP = 10240
SPT = NP // NS

TILE = 1000
NT = N // TILE

@functools.lru_cache(maxsize=1)
def _mesh():
    return plsc.VectorSubcoreMesh(core_axis_name="c", subcore_axis_name="s",
                                  num_cores=NC, num_subcores=NS)


def _sc_spmm(h, src, dst, zeros):
    @functools.partial(
        pl.kernel,
        out_type=jax.ShapeDtypeStruct((NC, NP, 128), jnp.float32),
        mesh=_mesh(),
        scratch_types=[
            pltpu.VMEM_SHARED((NP, 128), jnp.float32),
            pltpu.VMEM((ECH,), jnp.int32),
            pltpu.VMEM((ECH,), jnp.int32),
            pltpu.VMEM((ECH, 128), jnp.float32),
            pltpu.SemaphoreType.DMA,
        ],
    )
    def run(h_hbm, src_hbm, dst_hbm, z_hbm, out_hbm, acc, sidx, didx, rows, sem):
        cid = lax.axis_index("c")
        sid = lax.axis_index("s")
        r0 = sid * SPT
        pltpu.sync_copy(z_hbm.at[pl.ds(r0, SPT), :], acc.at[pl.ds(r0, SPT), :])
        plsc.subcore_barrier()
        base = (sid * NC + cid) * EPT

        def body(j, carry):
            off = base + j * ECH
            pltpu.sync_copy(src_hbm.at[pl.ds(off, ECH)], sidx)
            pltpu.sync_copy(dst_hbm.at[pl.ds(off, ECH)], didx)
            pltpu.async_copy(h_hbm.at[sidx], rows, sem).wait()
            pltpu.sync_copy(rows, acc.at[didx], add=True)
            return carry

        lax.fori_loop(0, NCHUNK, body, 0)
        plsc.subcore_barrier()
        r0 = sid * SPT
        pltpu.sync_copy(acc.at[pl.ds(r0, SPT), :], out_hbm.at[cid, pl.ds(r0, SPT), :])

    return run(h, src, dst, zeros)


def _onehot_batch(bid_col):
    io = lax.broadcasted_iota(jnp.int32, (bid_col.shape[0], G), 1)
    return jnp.where(bid_col.astype(jnp.int32) == io, 1.0, 0.0).astype(jnp.float32)


def _accumulate(ref, contrib):
    first = pl.program_id(0) == 0

    @pl.when(first)
    def _():
        ref[...] = contrib

    @pl.when(jnp.logical_not(first))
    def _():
        ref[...] = ref[...] + contrib


def _tc_prep(x, bidf, emb_pad):
    def body(x_ref, bid_ref, emb_ref, h_ref, ps_ref, cnt_ref):
        xb = x_ref[...]
        ids = xb[:, 0:1].astype(jnp.int32)
        io = lax.broadcasted_iota(jnp.int32, (TILE, SPAD), 1)
        oh_st = jnp.where(ids == io, 1.0, 0.0).astype(jnp.float32)
        emb = jnp.dot(oh_st, emb_ref[...], preferred_element_type=jnp.float32)
        h0 = jnp.concatenate([emb, xb[:, 1:65]], axis=1)
        h_ref[...] = h0
        ohb = _onehot_batch(bid_ref[...])
        dn = (((0,), (0,)), ((), ()))
        ps = lax.dot_general(ohb, h0, dn, preferred_element_type=jnp.float32)
        cnt = lax.dot_general(ohb, jnp.ones((TILE, 128), jnp.float32), dn,
                              preferred_element_type=jnp.float32)
        _accumulate(ps_ref, ps)
        _accumulate(cnt_ref, cnt)

    return pl.pallas_call(
        body,
        grid=(NT,),
        in_specs=[
            pl.BlockSpec((TILE, 65), lambda i: (i, 0)),
            pl.BlockSpec((TILE, 1), lambda i: (i, 0)),
            pl.BlockSpec((SPAD, 64), lambda i: (0, 0)),
        ],
        out_specs=[
            pl.BlockSpec((TILE, 128), lambda i: (i, 0)),
            pl.BlockSpec((G, 128), lambda i: (0, 0)),
            pl.BlockSpec((G, 128), lambda i: (0, 0)),
        ],
        out_shape=[
            jax.ShapeDtypeStruct((N, 128), jnp.float32),
            jax.ShapeDtypeStruct((G, 128), jnp.float32),
            jax.ShapeDtypeStruct((G, 128), jnp.float32),
        ],
    )(x, bidf, emb_pad)


def _gf_in(layer, psums, cmax, w5s):
    gf = psums[0] / cmax
    for l in range(2, layer + 1):
        W5a, W5b, b5 = w5s[l - 2]
        pooled = psums[l - 1] / cmax
        g = jnp.dot(gf, W5a, preferred_element_type=jnp.float32) \
            + jnp.dot(pooled, W5b, preferred_element_type=jnp.float32) + b5
        if l == 2:
            gf = jax.nn.relu(g)
        else:
            gf = gf + jax.nn.relu(g)
    return gf


def _tc_dense(layer, h, S, EA, bidf, psums, counts, wl, w5s, lin_Wp=None, lin_bp=None):
    W1, b1, W2, W3, W4d, W4s, W4e, W4g, b4 = wl
    nps = len(psums)

    def body(*refs):
        (h_ref, S_ref, EA_ref, bid_ref), rest = refs[:4], refs[4:]
        ps_refs = rest[:nps]
        cnt_ref = rest[nps]
        w_refs = rest[nps + 1:]
        if layer == 3:
            out_ref = refs[-1]
            w_refs = w_refs[:-1]
        else:
            hout_ref, psout_ref = refs[-2], refs[-1]
            w_refs = w_refs[:-2]

        (W1r, b1r, W2r, W3r, W4dr, W4sr, W4er, W4gr, b4r), w_rest = \
            w_refs[:9], w_refs[9:]

        cmax = jnp.maximum(cnt_ref[...], 1.0)
        w5vals = []
        k = 0
        for _ in range(layer - 1):
            w5vals.append((w_rest[k][...], w_rest[k + 1][...], w_rest[k + 2][...]))
            k += 3
        gf = _gf_in(layer, [r[...] for r in ps_refs], cmax, w5vals)

        hb = h_ref[...]
        Sb = S_ref[0, :, :] + S_ref[1, :, :]
        EAb = EA_ref[0, :, :] + EA_ref[1, :, :]
        deg = EAb[:, 16:17]
        rdeg = 1.0 / jnp.maximum(deg, 1.0)
        mask = jnp.where(deg > 0.0, 1.0, 0.0)
        Sm = Sb * rdeg
        eam = EAb[:, 0:16] * rdeg

        ohb = _onehot_batch(bid_ref[...])
        g_node = jnp.dot(ohb, gf, preferred_element_type=jnp.float32)

        agg = (jnp.dot(hb, W4dr[...], preferred_element_type=jnp.float32)
               + jnp.dot(g_node, W4gr[...], preferred_element_type=jnp.float32)
               + jnp.dot(Sm, W4sr[...], preferred_element_type=jnp.float32)
               + jnp.dot(eam, W4er[...], preferred_element_type=jnp.float32)
               + b4r[...]) * mask
        x_new = (jnp.dot(hb, W1r[...], preferred_element_type=jnp.float32) + b1r[...]
                 + jnp.dot(agg, W2r[...], preferred_element_type=jnp.float32)
                 + jnp.dot(g_node, W3r[...], preferred_element_type=jnp.float32))

        if layer == 1:
            h_out = jax.nn.relu(x_new)
        else:
            h_out = hb + jax.nn.relu(x_new)

        if layer == 3:
            linW_ref, linb_ref = w_rest[k], w_rest[k + 1]
            o = jnp.dot(h_out, linW_ref[...], preferred_element_type=jnp.float32) \
                + linb_ref[...]
            mu = o[:, 0:1]
            sg = jax.nn.softplus(o[:, 1:2])
            out_ref[...] = jnp.concatenate([mu, sg], axis=1)
        else:
            hout_ref[...] = h_out
            dn = (((0,), (0,)), ((), ()))
            _accumulate(psout_ref,
                        lax.dot_general(ohb, x_new, dn,
                                        preferred_element_type=jnp.float32))

    full = lambda a, b: pl.BlockSpec((a, b), lambda i: (0, 0))
    in_specs = [
        pl.BlockSpec((TILE, 128), lambda i: (i, 0)),
        pl.BlockSpec((NC, TILE, 128), lambda i: (0, i, 0)),
        pl.BlockSpec((NC, TILE, 128), lambda i: (0, i, 0)),
        pl.BlockSpec((TILE, 1), lambda i: (i, 0)),
    ]
    args = [h, S, EA, bidf] + list(psums) + [counts]
    in_specs += [full(G, 128)] * nps + [full(G, 128)]
    wargs = [W1, b1, W2, W3, W4d, W4s, W4e, W4g, b4]
    wspecs = [full(128, 128), full(1, 128), full(128, 128), full(128, 128),
              full(128, 128), full(128, 128), full(16, 128), full(128, 128),
              full(1, 128)]
    for (W5a, W5b, b5) in w5s[:layer - 1]:
        wargs += [W5a, W5b, b5]
        wspecs += [full(128, 128), full(128, 128), full(1, 128)]
    if layer == 3:
        wargs += [lin_Wp, lin_bp]
        wspecs += [full(128, 128), full(1, 128)]
        out_specs = [pl.BlockSpec((TILE, 2), lambda i: (i, 0))]
        out_shape = [jax.ShapeDtypeStruct((N, 2), jnp.float32)]
    else:
        out_specs = [pl.BlockSpec((TILE, 128), lambda i: (i, 0)),
                     pl.BlockSpec((G, 128), lambda i: (0, 0))]
        out_shape = [jax.ShapeDtypeStruct((N, 128), jnp.float32),
                     jax.ShapeDtypeStruct((G, 128), jnp.float32)]

    return pl.pallas_call(
        body,
        grid=(NT,),
        in_specs=in_specs + wspecs,
        out_specs=out_specs,
        out_shape=out_shape,
    )(*(args + wargs))


def kernel(x, edge_index, edge_attr, batch_id, emb_table, params, lin_W, lin_b):
    f32 = jnp.float32
    src = edge_index[0]
    dst = edge_index[1]
    bidf = batch_id.astype(f32).reshape(N, 1)
    ea_pad = jnp.concatenate(
        [edge_attr, jnp.ones((E, 1), f32), jnp.zeros((E, 111), f32)], axis=1)
    eidx = jnp.arange(E, dtype=jnp.int32)
    zeros128 = jnp.zeros((NP, 128), f32)
    emb_pad = jnp.zeros((SPAD, 64), f32).at[:STATIONS].set(emb_table)
    lin_Wp = jnp.zeros((128, 128), f32).at[:, :2].set(lin_W)
    lin_bp = jnp.zeros((1, 128), f32).at[0, :2].set(lin_b)

    def wl(p):
        W4 = p['W4']
        return (p['W1'], p['b1'].reshape(1, 128), p['W2'], p['W3'],
                W4[0:128], W4[128:256], W4[256:272], W4[272:400],
                p['b4'].reshape(1, 128))

    w5s = [(p['W5'][0:128], p['W5'][128:256], p['b5'].reshape(1, 128))
           for p in params]

    EA = _sc_spmm(ea_pad, eidx, dst, zeros128)
    h0, psum0, counts = _tc_prep(x, bidf, emb_pad)

    S1 = _sc_spmm(h0, src, dst, zeros128)
    h1, psum1 = _tc_dense(1, h0, S1, EA, bidf, [psum0], counts, wl(params[0]), w5s)
    S2 = _sc_spmm(h1, src, dst, zeros128)
    h2, psum2 = _tc_dense(2, h1, S2, EA, bidf, [psum0, psum1], counts,
                          wl(params[1]), w5s)
    S3 = _sc_spmm(h2, src, dst, zeros128)
    out = _tc_dense(3, h2, S3, EA, bidf, [psum0, psum1, psum2], counts,
                    wl(params[2]), w5s, lin_Wp, lin_bp)
    return out[0]
